# baseline (speedup 1.0000x reference)
import functools

import jax
import jax.numpy as jnp
from jax import lax
from jax.experimental import pallas as pl
from jax.experimental.pallas import tpu as pltpu
from jax.experimental.pallas import tpu_sc as plsc

N = 320000
D = 128
NSEG = 10000

NC = 2
NS = 16
L = 16

NSEG_PAD = 10240
SLICE = NSEG_PAD // NS
MV_B = 12800
CHUNKS = ((0, 217600), (217600, 64000), (281600, 38400))

ZUNROLL = 8
SUNROLL = 5


def _matvec_body(x_ref, w_ref, b_ref, o_ref):
    xb = x_ref[0]
    s = jax.lax.dot_general(
        w_ref[...], xb, (((1,), (1,)), ((), ())),
        preferred_element_type=jnp.float32,
    )
    o_ref[...] = (s + b_ref[0, 0]).reshape(1, 1, MV_B)


def _matvec_part(x3, w_row, b11, blk0, nblk):
    out = pl.pallas_call(
        _matvec_body,
        grid=(nblk,),
        in_specs=[
            pl.BlockSpec((1, MV_B, D), lambda i: (i + blk0, 0, 0)),
            pl.BlockSpec((1, D), lambda i: (0, 0)),
            pl.BlockSpec((1, 1), lambda i: (0, 0), memory_space=pltpu.SMEM),
        ],
        out_specs=pl.BlockSpec((1, 1, MV_B), lambda i: (i, 0, 0)),
        out_shape=jax.ShapeDtypeStruct((nblk, 1, MV_B), jnp.float32),
    )(x3, w_row, b11)
    return out.reshape(nblk * MV_B)


def _make_segsum(off, total):
    chunk = total // (NC * NS)

    def body(batch_hbm, y_hbm, out_hbm, idx_v, y_v, acc, stage, red_v, acc2,
             sem_i, sem_y):
        c = lax.axis_index("c")
        s = lax.axis_index("s")
        wid = s * NC + c
        base = wid * chunk

        cp_i = pltpu.async_copy(batch_hbm.at[pl.ds(off + base, chunk)], idx_v, sem_i)
        cp_y = pltpu.async_copy(y_hbm.at[pl.ds(base, chunk)], y_v, sem_y)

        zero = jnp.zeros((L,), jnp.float32)

        def zbody(i, _):
            for u in range(ZUNROLL):
                acc[pl.ds((i * ZUNROLL + u) * L, L)] = zero
            return 0

        lax.fori_loop(0, NSEG_PAD // L // ZUNROLL, zbody, 0)
        cp_i.wait()
        cp_y.wait()

        def sbody(i, _):
            for u in range(SUNROLL):
                sl = pl.ds((i * SUNROLL + u) * L, L)
                plsc.addupdate_scatter(acc, [idx_v[sl]], y_v[sl])
            return 0

        lax.fori_loop(0, chunk // L // SUNROLL, sbody, 0)

        pltpu.sync_copy(acc, stage.at[s])
        plsc.subcore_barrier()
        pltpu.sync_copy(stage.at[:, pl.ds(s * SLICE, SLICE)], red_v)

        def rbody(j, _):
            sl = pl.ds(j * L, L)
            v = red_v[0, sl]
            for k in range(1, NS):
                v = v + red_v[k, sl]
            acc2[sl] = v
            return 0

        lax.fori_loop(0, SLICE // L, rbody, 0)

        pltpu.sync_copy(acc2, out_hbm.at[pl.ds(c * NSEG_PAD + s * SLICE, SLICE)])

    return pl.kernel(
        body,
        out_type=jax.ShapeDtypeStruct((NC * NSEG_PAD,), jnp.float32),
        mesh=plsc.VectorSubcoreMesh(core_axis_name="c", subcore_axis_name="s"),
        compiler_params=pltpu.CompilerParams(needs_layout_passes=False),
        scratch_types=[
            pltpu.VMEM((chunk,), jnp.int32),
            pltpu.VMEM((chunk,), jnp.float32),
            pltpu.VMEM((NSEG_PAD,), jnp.float32),
            pltpu.VMEM_SHARED((NS, NSEG_PAD), jnp.float32),
            pltpu.VMEM((NS, SLICE), jnp.float32),
            pltpu.VMEM((SLICE,), jnp.float32),
            pltpu.SemaphoreType.DMA,
            pltpu.SemaphoreType.DMA,
        ],
    )


@functools.cache
def _segsum_kernels():
    return tuple(_make_segsum(off, total) for off, total in CHUNKS)


def kernel(atom_batch, x, W, b):
    ids = atom_batch.astype(jnp.int32)
    w_row = W.reshape(1, D).astype(jnp.float32)
    b11 = b.reshape(1, 1).astype(jnp.float32)
    x3 = x.reshape(N // MV_B, MV_B, D)
    segsums = _segsum_kernels()
    acc = None
    blk0 = 0
    for (off, total), segsum in zip(CHUNKS, segsums):
        nblk = total // MV_B
        y_part = _matvec_part(x3, w_row, b11, blk0, nblk)
        blk0 += nblk
        partials = segsum(ids, y_part).reshape(NC, NSEG_PAD)
        part = partials[0] + partials[1]
        acc = part if acc is None else acc + part
    return acc[:NSEG]


# speedup vs baseline: 1.0745x; 1.0745x over previous
import functools

import jax
import jax.numpy as jnp
from jax import lax
from jax.experimental import pallas as pl
from jax.experimental.pallas import tpu as pltpu
from jax.experimental.pallas import tpu_sc as plsc

N = 320000
D = 128
NSEG = 10000

NC = 2
NS = 16
L = 16

NSEG_PAD = 10240
SLICE = NSEG_PAD // NS
MV_B = 12800
CHUNKS = ((0, 256000), (256000, 64000))

ZUNROLL = 8
SUNROLL = 5


def _matvec_body(x_ref, w_ref, b_ref, o_ref):
    xb = x_ref[0]
    s = jax.lax.dot_general(
        w_ref[...], xb, (((1,), (1,)), ((), ())),
        preferred_element_type=jnp.float32,
    )
    o_ref[...] = (s + b_ref[0, 0]).reshape(1, 1, MV_B)


def _matvec_part(x3, w_row, b11, blk0, nblk):
    out = pl.pallas_call(
        _matvec_body,
        grid=(nblk,),
        in_specs=[
            pl.BlockSpec((1, MV_B, D), lambda i: (i + blk0, 0, 0)),
            pl.BlockSpec((1, D), lambda i: (0, 0)),
            pl.BlockSpec((1, 1), lambda i: (0, 0), memory_space=pltpu.SMEM),
        ],
        out_specs=pl.BlockSpec((1, 1, MV_B), lambda i: (i, 0, 0)),
        out_shape=jax.ShapeDtypeStruct((nblk, 1, MV_B), jnp.float32),
    )(x3, w_row, b11)
    return out.reshape(nblk * MV_B)


def _make_segsum(off, total):
    chunk = total // (NC * NS)

    def body(batch_hbm, y_hbm, out_hbm, idx_v, y_v, acc, stage, red_v, acc2,
             sem_i, sem_y):
        c = lax.axis_index("c")
        s = lax.axis_index("s")
        wid = s * NC + c
        base = wid * chunk

        cp_i = pltpu.async_copy(batch_hbm.at[pl.ds(off + base, chunk)], idx_v, sem_i)
        cp_y = pltpu.async_copy(y_hbm.at[pl.ds(base, chunk)], y_v, sem_y)

        zero = jnp.zeros((L,), jnp.float32)

        def zbody(i, _):
            for u in range(ZUNROLL):
                acc[pl.ds((i * ZUNROLL + u) * L, L)] = zero
            return 0

        lax.fori_loop(0, NSEG_PAD // L // ZUNROLL, zbody, 0)
        cp_i.wait()
        cp_y.wait()

        def sbody(i, _):
            for u in range(SUNROLL):
                sl = pl.ds((i * SUNROLL + u) * L, L)
                plsc.addupdate_scatter(acc, [idx_v[sl]], y_v[sl])
            return 0

        lax.fori_loop(0, chunk // L // SUNROLL, sbody, 0)

        pltpu.sync_copy(acc, stage.at[s])
        plsc.subcore_barrier()
        pltpu.sync_copy(stage.at[:, pl.ds(s * SLICE, SLICE)], red_v)

        def rbody(j, _):
            sl = pl.ds(j * L, L)
            v = red_v[0, sl]
            for k in range(1, NS):
                v = v + red_v[k, sl]
            acc2[sl] = v
            return 0

        lax.fori_loop(0, SLICE // L, rbody, 0)

        pltpu.sync_copy(acc2, out_hbm.at[pl.ds(c * NSEG_PAD + s * SLICE, SLICE)])

    return pl.kernel(
        body,
        out_type=jax.ShapeDtypeStruct((NC * NSEG_PAD,), jnp.float32),
        mesh=plsc.VectorSubcoreMesh(core_axis_name="c", subcore_axis_name="s"),
        compiler_params=pltpu.CompilerParams(needs_layout_passes=False),
        scratch_types=[
            pltpu.VMEM((chunk,), jnp.int32),
            pltpu.VMEM((chunk,), jnp.float32),
            pltpu.VMEM((NSEG_PAD,), jnp.float32),
            pltpu.VMEM_SHARED((NS, NSEG_PAD), jnp.float32),
            pltpu.VMEM((NS, SLICE), jnp.float32),
            pltpu.VMEM((SLICE,), jnp.float32),
            pltpu.SemaphoreType.DMA,
            pltpu.SemaphoreType.DMA,
        ],
    )


@functools.cache
def _segsum_kernels():
    return tuple(_make_segsum(off, total) for off, total in CHUNKS)


def kernel(atom_batch, x, W, b):
    ids = atom_batch.astype(jnp.int32)
    w_row = W.reshape(1, D).astype(jnp.float32)
    b11 = b.reshape(1, 1).astype(jnp.float32)
    x3 = x.reshape(N // MV_B, MV_B, D)
    segsums = _segsum_kernels()
    acc = None
    blk0 = 0
    for (off, total), segsum in zip(CHUNKS, segsums):
        nblk = total // MV_B
        y_part = _matvec_part(x3, w_row, b11, blk0, nblk)
        blk0 += nblk
        partials = segsum(ids, y_part).reshape(NC, NSEG_PAD)
        part = partials[0] + partials[1]
        acc = part if acc is None else acc + part
    return acc[:NSEG]
